# baseline (speedup 1.0000x reference)
import functools

import jax
import jax.numpy as jnp
from jax import lax
from jax.experimental import pallas as pl
from jax.experimental.pallas import tpu as pltpu
from jax.experimental.pallas import tpu_sc as plsc

N = 10000
E = 320000
F_IN = 128
F_ALL = 96
F_PAD = 128
F_OUT = 32

NC = 2
NS = 16
NW = NC * NS
CH = 128

N_PAD = 10240
ROWS_SUB = N_PAD // NS
EW = ((E // NW + 8 * CH - 1) // (8 * CH)) * (8 * CH)
K = EW // CH
E_PAD = EW * NW

BN = 1280
GRID = N_PAD // BN

_mesh = plsc.VectorSubcoreMesh(core_axis_name="c", subcore_axis_name="s")


def _full16(v):
    return jnp.full((16,), v, dtype=jnp.int32)


@functools.partial(
    pl.kernel,
    mesh=_mesh,
    out_type=[
        jax.ShapeDtypeStruct((N_PAD, F_PAD), jnp.float32),
        jax.ShapeDtypeStruct((N_PAD, F_PAD), jnp.float32),
    ],
    scratch_types=[
        pltpu.VMEM((16, CH), jnp.int32),
        pltpu.VMEM((16, CH), jnp.float32),
        pltpu.VMEM((CH, F_PAD), jnp.float32),
        pltpu.VMEM((CH, F_PAD), jnp.float32),
        pltpu.VMEM_SHARED((N_PAD, F_PAD), jnp.float32),
        pltpu.SemaphoreType.DMA,
        pltpu.SemaphoreType.DMA,
    ],
)
def _sc_degree(dst_hbm, w_hbm, out0_hbm, out1_hbm, dst_v, w_v, rows0_v,
               rows1_v, acc_sh, ssem0, ssem1):
    c = lax.axis_index("c")
    s = lax.axis_index("s")
    wid = c * NS + s

    for rv in (rows0_v, rows1_v):
        def _z(r, _):
            for f in range(F_PAD // 16):
                rv[r, pl.ds(f * 16, 16)] = jnp.zeros((16,), jnp.float32)
            return 0
        lax.fori_loop(0, CH, _z, 0, unroll=4)
    for t in range(ROWS_SUB // CH):
        pltpu.sync_copy(rows0_v, acc_sh.at[pl.ds(s * ROWS_SUB + t * CH, CH)])
    plsc.subcore_barrier()

    def _block(b, _):
        base = pl.multiple_of(wid * K + b * 16, 8)
        pltpu.sync_copy(dst_hbm.at[pl.ds(base, 16)], dst_v)
        pltpu.sync_copy(w_hbm.at[pl.ds(base, 16)], w_v)

        def _pair(p, _):
            bufs = ((rows0_v, ssem0), (rows1_v, ssem1))
            for h in range(2):
                j = 2 * p + h
                rows_v, ssem = bufs[h]
                nrows_v, nssem = bufs[1 - h]

                @pl.when(j >= 1)
                def _():
                    pltpu.make_async_copy(
                        nrows_v, acc_sh.at[dst_v.at[j - 1]], nssem).wait()

                def _build(t, _):
                    w16 = w_v[j, pl.ds(t * 16, 16)]
                    for el in range(16):
                        rows_v[t * 16 + el, pl.ds(0, 16)] = jnp.full(
                            (16,), w16[el], dtype=jnp.float32)
                    return 0
                lax.fori_loop(0, CH // 16, _build, 0)
                pltpu.async_copy(rows_v, acc_sh.at[dst_v.at[j]], ssem,
                                 add=True)
            return 0
        lax.fori_loop(0, 8, _pair, 0)
        pltpu.make_async_copy(rows1_v, acc_sh.at[dst_v.at[15]],
                              ssem1).wait()
        return 0
    lax.fori_loop(0, K // 16, _block, 0)
    plsc.subcore_barrier()

    for t in range(ROWS_SUB // CH):
        base = s * ROWS_SUB + t * CH
        pltpu.sync_copy(acc_sh.at[pl.ds(base, CH)], rows0_v)

        @pl.when(c == 0)
        def _():
            pltpu.sync_copy(rows0_v, out0_hbm.at[pl.ds(base, CH)])

        @pl.when(c == 1)
        def _():
            pltpu.sync_copy(rows0_v, out1_hbm.at[pl.ds(base, CH)])


BI = 16
NB = K // BI


@functools.partial(
    pl.kernel,
    mesh=_mesh,
    out_type=[
        jax.ShapeDtypeStruct((N_PAD, F_PAD), jnp.float32),
        jax.ShapeDtypeStruct((N_PAD, F_PAD), jnp.float32),
    ],
    scratch_types=[
        pltpu.VMEM((BI, CH), jnp.int32),
        pltpu.VMEM((BI, CH), jnp.int32),
        pltpu.VMEM((BI, CH), jnp.float32),
        pltpu.VMEM((CH, F_PAD), jnp.float32),
        pltpu.VMEM((CH, F_PAD), jnp.float32),
        pltpu.VMEM_SHARED((N_PAD, F_PAD), jnp.float32),
        pltpu.SemaphoreType.DMA,
        pltpu.SemaphoreType.DMA,
    ],
)
def _sc_scatter(g_hbm, src_hbm, dst_hbm, w_hbm, out0_hbm, out1_hbm,
                src_v, dst_v, w_v, rows0_v, rows1_v, acc_sh,
                gsem0, gsem1):
    c = lax.axis_index("c")
    s = lax.axis_index("s")
    wid = c * NS + s

    def _z(r, _):
        for f in range(F_PAD // 16):
            rows0_v[r, pl.ds(f * 16, 16)] = jnp.zeros((16,), jnp.float32)
        return 0
    lax.fori_loop(0, CH, _z, 0, unroll=4)
    for t in range(ROWS_SUB // CH):
        pltpu.sync_copy(rows0_v, acc_sh.at[pl.ds(s * ROWS_SUB + t * CH, CH)])
    plsc.subcore_barrier()

    def _scale(rows_v, j):
        def _grp(t, _):
            w16 = w_v[j, pl.ds(t * 16, 16)]
            for el in range(16):
                wsp = jnp.full((16,), w16[el], dtype=jnp.float32)
                e = t * 16 + el
                for f in range(F_ALL // 16):
                    sl = pl.ds(f * 16, 16)
                    rows_v[e, sl] = rows_v[e, sl] * wsp
            return 0
        lax.fori_loop(0, CH // 16, _grp, 0)

    def _block(b, _):
        base = pl.multiple_of(wid * K + b * BI, 8)
        pltpu.sync_copy(src_hbm.at[pl.ds(base, BI)], src_v)
        pltpu.sync_copy(dst_hbm.at[pl.ds(base, BI)], dst_v)
        pltpu.sync_copy(w_hbm.at[pl.ds(base, BI)], w_v)
        pltpu.async_copy(g_hbm.at[src_v.at[0]], rows0_v, gsem0)

        def _pair(p, _):
            bufs = ((rows0_v, gsem0), (rows1_v, gsem1))
            for h in range(2):
                j = 2 * p + h
                rows_v, gsem = bufs[h]
                nrows_v, ngsem = bufs[1 - h]

                pltpu.make_async_copy(g_hbm.at[src_v.at[j]], rows_v,
                                      gsem).wait()

                @pl.when(j + 1 < BI)
                def _():
                    pltpu.async_copy(g_hbm.at[src_v.at[j + 1]], nrows_v,
                                     ngsem)
                _scale(rows_v, j)
                pltpu.sync_copy(rows_v, acc_sh.at[dst_v.at[j]], add=True)
            return 0
        lax.fori_loop(0, BI // 2, _pair, 0)
        return 0
    lax.fori_loop(0, NB, _block, 0)
    plsc.subcore_barrier()

    for t in range(ROWS_SUB // CH):
        base = s * ROWS_SUB + t * CH
        pltpu.sync_copy(acc_sh.at[pl.ds(base, CH)], rows0_v)

        @pl.when(c == 0)
        def _():
            pltpu.sync_copy(rows0_v, out0_hbm.at[pl.ds(base, CH)])

        @pl.when(c == 1)
        def _():
            pltpu.sync_copy(rows0_v, out1_hbm.at[pl.ds(base, CH)])


def _mm_body(x_ref, w_ref, o_ref):
    o_ref[...] = lax.dot_general(
        x_ref[...], w_ref[...], (((1,), (0,)), ((), ())),
        preferred_element_type=jnp.float32)


def _matmul(x, w_all):
    return pl.pallas_call(
        _mm_body,
        grid=(GRID,),
        in_specs=[
            pl.BlockSpec((BN, F_IN), lambda i: (i, 0)),
            pl.BlockSpec((F_IN, F_PAD), lambda i: (0, 0)),
        ],
        out_specs=pl.BlockSpec((BN, F_PAD), lambda i: (i, 0)),
        out_shape=jax.ShapeDtypeStruct((N_PAD, F_PAD), jnp.float32),
    )(x, w_all)


def _scale_body(d0_ref, d1_ref, h_ref, g_ref, dv_ref):
    deg = d0_ref[:, 0:1] + d1_ref[:, 0:1] + 1.0
    dinv = lax.rsqrt(deg)
    g_ref[...] = dinv * h_ref[...]
    dv_ref[...] = jnp.broadcast_to(dinv, (BN, 16))


def _scale(d0, d1, h):
    return pl.pallas_call(
        _scale_body,
        grid=(GRID,),
        in_specs=[
            pl.BlockSpec((BN, F_PAD), lambda i: (i, 0)),
            pl.BlockSpec((BN, F_PAD), lambda i: (i, 0)),
            pl.BlockSpec((BN, F_PAD), lambda i: (i, 0)),
        ],
        out_specs=[
            pl.BlockSpec((BN, F_PAD), lambda i: (i, 0)),
            pl.BlockSpec((BN, 16), lambda i: (i, 0)),
        ],
        out_shape=[
            jax.ShapeDtypeStruct((N_PAD, F_PAD), jnp.float32),
            jax.ShapeDtypeStruct((N_PAD, 16), jnp.float32),
        ],
    )(d0, d1, h)


def _dot(a, b):
    return lax.dot_general(a, b, (((1,), (0,)), ((), ())),
                           preferred_element_type=jnp.float32)


def _gru_body(p0_ref, p1_ref, g_ref, dv_ref, h_ref, bc_ref,
              lt_ref, lb_ref, lbias_ref, lw_ref, lb2_ref,
              y_ref, hn_ref):
    dinv = dv_ref[:, 0:1]
    conv = dinv * (p0_ref[:, 0:F_ALL] + p1_ref[:, 0:F_ALL]
                   + g_ref[:, 0:F_ALL]) + bc_ref[...]
    H = h_ref[...]
    cz = conv[:, 0:32]
    cr = conv[:, 32:64]
    ch = conv[:, 64:96]
    lt = lt_ref[...]
    lb = lb_ref[...]
    lbias = lbias_ref[...]
    Z = jax.nn.sigmoid(_dot(cz, lt[:, 0:32]) + _dot(H, lb[:, 0:32])
                       + lbias[:, 0:32])
    R = jax.nn.sigmoid(_dot(cr, lt[:, 32:64]) + _dot(H, lb[:, 32:64])
                       + lbias[:, 32:64])
    Ht = jnp.tanh(_dot(ch, lt[:, 64:96]) + _dot(H * R, lb[:, 64:96])
                  + lbias[:, 64:96])
    Hn = Z * H + (1.0 - Z) * Ht
    hn_ref[...] = Hn
    y_ref[...] = _dot(jnp.maximum(Hn, 0.0), lw_ref[...]) + lb2_ref[...]


def _gru(p0, p1, g, dv, Hp, bcat, lt, lb, lbias, lin_w, lin_b):
    return pl.pallas_call(
        _gru_body,
        grid=(GRID,),
        in_specs=[
            pl.BlockSpec((BN, F_PAD), lambda i: (i, 0)),
            pl.BlockSpec((BN, F_PAD), lambda i: (i, 0)),
            pl.BlockSpec((BN, F_PAD), lambda i: (i, 0)),
            pl.BlockSpec((BN, 16), lambda i: (i, 0)),
            pl.BlockSpec((BN, F_OUT), lambda i: (i, 0)),
            pl.BlockSpec((1, F_ALL), lambda i: (0, 0)),
            pl.BlockSpec((F_OUT, F_ALL), lambda i: (0, 0)),
            pl.BlockSpec((F_OUT, F_ALL), lambda i: (0, 0)),
            pl.BlockSpec((1, F_ALL), lambda i: (0, 0)),
            pl.BlockSpec((F_OUT, 1), lambda i: (0, 0)),
            pl.BlockSpec((1, 1), lambda i: (0, 0)),
        ],
        out_specs=[
            pl.BlockSpec((BN, 1), lambda i: (i, 0)),
            pl.BlockSpec((BN, F_OUT), lambda i: (i, 0)),
        ],
        out_shape=[
            jax.ShapeDtypeStruct((N_PAD, 1), jnp.float32),
            jax.ShapeDtypeStruct((N_PAD, F_OUT), jnp.float32),
        ],
    )(p0, p1, g, dv, Hp, bcat, lt, lb, lbias, lin_w, lin_b)


def kernel(node_feat, edge_index, edge_weight, hidden_state,
           Wz, bz, Wr, br, Wh, bh,
           Lz_w, Lz_b, Lr_w, Lr_b, Lh_w, Lh_b, lin_w, lin_b):
    x = jnp.pad(node_feat, ((0, N_PAD - N), (0, 0)))
    w_all = jnp.pad(jnp.concatenate([Wz, Wr, Wh], axis=1),
                    ((0, 0), (0, F_PAD - F_ALL)))

    src = jnp.pad(edge_index[0], (0, E_PAD - E)).reshape(E_PAD // CH, CH)
    dst = jnp.pad(edge_index[1], (0, E_PAD - E)).reshape(E_PAD // CH, CH)
    w = jnp.pad(edge_weight, (0, E_PAD - E)).reshape(E_PAD // CH, CH)

    h = _matmul(x, w_all)
    deg0, deg1 = _sc_degree(dst, w)
    g, dv = _scale(deg0, deg1, h)
    acc0, acc1 = _sc_scatter(g, src, dst, w)

    Hp = jnp.pad(hidden_state, ((0, N_PAD - N), (0, 0)))
    bcat = jnp.concatenate([bz, br, bh])[None, :]
    lt = jnp.concatenate([Lz_w[:F_OUT], Lr_w[:F_OUT], Lh_w[:F_OUT]], axis=1)
    lb = jnp.concatenate([Lz_w[F_OUT:], Lr_w[F_OUT:], Lh_w[F_OUT:]], axis=1)
    lbias = jnp.concatenate([Lz_b, Lr_b, Lh_b])[None, :]

    y_p, hn_p = _gru(acc0, acc1, g, dv, Hp, bcat, lt, lb, lbias,
                     lin_w, lin_b[None, :])
    return (y_p[:N], hn_p[:N])


# speedup vs baseline: 1.0476x; 1.0476x over previous
import functools

import jax
import jax.numpy as jnp
from jax import lax
from jax.experimental import pallas as pl
from jax.experimental.pallas import tpu as pltpu
from jax.experimental.pallas import tpu_sc as plsc

N = 10000
E = 320000
F_IN = 128
F_ALL = 96
F_PAD = 128
F_OUT = 32

NC = 2
NS = 16
NW = NC * NS
CH = 128

N_PAD = 10240
ROWS_SUB = N_PAD // NS
EW = ((E // NW + 8 * CH - 1) // (8 * CH)) * (8 * CH)
K = EW // CH
E_PAD = EW * NW

BN = 1280
GRID = N_PAD // BN

_mesh = plsc.VectorSubcoreMesh(core_axis_name="c", subcore_axis_name="s")


def _full16(v):
    return jnp.full((16,), v, dtype=jnp.int32)


@functools.partial(
    pl.kernel,
    mesh=_mesh,
    out_type=jax.ShapeDtypeStruct((NC, N_PAD, F_PAD), jnp.float32),
    scratch_types=[
        pltpu.VMEM((16, CH), jnp.int32),
        pltpu.VMEM((16, CH), jnp.float32),
        pltpu.VMEM((CH, F_PAD), jnp.float32),
        pltpu.VMEM((CH, F_PAD), jnp.float32),
        pltpu.VMEM_SHARED((N_PAD, F_PAD), jnp.float32),
        pltpu.SemaphoreType.DMA,
        pltpu.SemaphoreType.DMA,
    ],
)
def _sc_degree(dst_hbm, w_hbm, out_hbm, dst_v, w_v, rows0_v, rows1_v,
               acc_sh, ssem0, ssem1):
    c = lax.axis_index("c")
    s = lax.axis_index("s")
    wid = c * NS + s

    for rv in (rows0_v, rows1_v):
        def _z(r, _):
            for f in range(F_PAD // 16):
                rv[r, pl.ds(f * 16, 16)] = jnp.zeros((16,), jnp.float32)
            return 0
        lax.fori_loop(0, CH, _z, 0, unroll=4)
    for t in range(ROWS_SUB // CH):
        pltpu.sync_copy(rows0_v, acc_sh.at[pl.ds(s * ROWS_SUB + t * CH, CH)])
    plsc.subcore_barrier()

    def _block(b, _):
        base = pl.multiple_of(wid * K + b * 16, 8)
        pltpu.sync_copy(dst_hbm.at[pl.ds(base, 16)], dst_v)
        pltpu.sync_copy(w_hbm.at[pl.ds(base, 16)], w_v)

        def _pair(p, _):
            bufs = ((rows0_v, ssem0), (rows1_v, ssem1))
            for h in range(2):
                j = 2 * p + h
                rows_v, ssem = bufs[h]
                nrows_v, nssem = bufs[1 - h]

                @pl.when(j >= 1)
                def _():
                    pltpu.make_async_copy(
                        nrows_v, acc_sh.at[dst_v.at[j - 1]], nssem).wait()

                def _build(t, _):
                    w16 = w_v[j, pl.ds(t * 16, 16)]
                    for el in range(16):
                        rows_v[t * 16 + el, pl.ds(0, 16)] = jnp.full(
                            (16,), w16[el], dtype=jnp.float32)
                    return 0
                lax.fori_loop(0, CH // 16, _build, 0)
                pltpu.async_copy(rows_v, acc_sh.at[dst_v.at[j]], ssem,
                                 add=True)
            return 0
        lax.fori_loop(0, 8, _pair, 0)
        pltpu.make_async_copy(rows1_v, acc_sh.at[dst_v.at[15]],
                              ssem1).wait()
        return 0
    lax.fori_loop(0, K // 16, _block, 0)
    plsc.subcore_barrier()

    for t in range(ROWS_SUB // CH):
        base = s * ROWS_SUB + t * CH
        pltpu.sync_copy(acc_sh.at[pl.ds(base, CH)], rows0_v)
        pltpu.sync_copy(rows0_v, out_hbm.at[c, pl.ds(base, CH)])


BI = 16
NB = K // BI


@functools.partial(
    pl.kernel,
    mesh=_mesh,
    out_type=jax.ShapeDtypeStruct((NC, N_PAD, F_PAD), jnp.float32),
    scratch_types=[
        pltpu.VMEM((BI, CH), jnp.int32),
        pltpu.VMEM((BI, CH), jnp.int32),
        pltpu.VMEM((BI, CH), jnp.float32),
        pltpu.VMEM((CH, F_PAD), jnp.float32),
        pltpu.VMEM((CH, F_PAD), jnp.float32),
        pltpu.VMEM_SHARED((N_PAD, F_PAD), jnp.float32),
        pltpu.SemaphoreType.DMA,
        pltpu.SemaphoreType.DMA,
    ],
)
def _sc_scatter(g_hbm, src_hbm, dst_hbm, w_hbm, out_hbm,
                src_v, dst_v, w_v, rows0_v, rows1_v, acc_sh,
                gsem0, gsem1):
    c = lax.axis_index("c")
    s = lax.axis_index("s")
    wid = c * NS + s

    def _z(r, _):
        for f in range(F_PAD // 16):
            rows0_v[r, pl.ds(f * 16, 16)] = jnp.zeros((16,), jnp.float32)
        return 0
    lax.fori_loop(0, CH, _z, 0, unroll=4)
    for t in range(ROWS_SUB // CH):
        pltpu.sync_copy(rows0_v, acc_sh.at[pl.ds(s * ROWS_SUB + t * CH, CH)])
    plsc.subcore_barrier()

    def _scale(rows_v, j):
        def _grp(t, _):
            w16 = w_v[j, pl.ds(t * 16, 16)]
            for el in range(16):
                wsp = jnp.full((16,), w16[el], dtype=jnp.float32)
                e = t * 16 + el
                for f in range(F_ALL // 16):
                    sl = pl.ds(f * 16, 16)
                    rows_v[e, sl] = rows_v[e, sl] * wsp
            return 0
        lax.fori_loop(0, CH // 16, _grp, 0)

    def _block(b, _):
        base = pl.multiple_of(wid * K + b * BI, 8)
        pltpu.sync_copy(src_hbm.at[pl.ds(base, BI)], src_v)
        pltpu.sync_copy(dst_hbm.at[pl.ds(base, BI)], dst_v)
        pltpu.sync_copy(w_hbm.at[pl.ds(base, BI)], w_v)
        pltpu.async_copy(g_hbm.at[src_v.at[0]], rows0_v, gsem0)

        def _pair(p, _):
            bufs = ((rows0_v, gsem0), (rows1_v, gsem1))
            for h in range(2):
                j = 2 * p + h
                rows_v, gsem = bufs[h]
                nrows_v, ngsem = bufs[1 - h]

                pltpu.make_async_copy(g_hbm.at[src_v.at[j]], rows_v,
                                      gsem).wait()

                @pl.when(j + 1 < BI)
                def _():
                    pltpu.async_copy(g_hbm.at[src_v.at[j + 1]], nrows_v,
                                     ngsem)
                _scale(rows_v, j)
                pltpu.sync_copy(rows_v, acc_sh.at[dst_v.at[j]], add=True)
            return 0
        lax.fori_loop(0, BI // 2, _pair, 0)
        return 0
    lax.fori_loop(0, NB, _block, 0)
    plsc.subcore_barrier()

    for t in range(ROWS_SUB // CH):
        base = s * ROWS_SUB + t * CH
        pltpu.sync_copy(acc_sh.at[pl.ds(base, CH)], rows0_v)
        pltpu.sync_copy(rows0_v, out_hbm.at[c, pl.ds(base, CH)])


def _mm_body(x_ref, w_ref, o_ref):
    o_ref[...] = lax.dot_general(
        x_ref[...], w_ref[...], (((1,), (0,)), ((), ())),
        preferred_element_type=jnp.float32)


def _matmul(x, w_all):
    return pl.pallas_call(
        _mm_body,
        grid=(GRID,),
        in_specs=[
            pl.BlockSpec((BN, F_IN), lambda i: (i, 0)),
            pl.BlockSpec((F_IN, F_PAD), lambda i: (0, 0)),
        ],
        out_specs=pl.BlockSpec((BN, F_PAD), lambda i: (i, 0)),
        out_shape=jax.ShapeDtypeStruct((N_PAD, F_PAD), jnp.float32),
    )(x, w_all)


def _scale_body(d0_ref, d1_ref, h_ref, g_ref, dv_ref):
    deg = d0_ref[:, 0:1] + d1_ref[:, 0:1] + 1.0
    dinv = lax.rsqrt(deg)
    g_ref[...] = dinv * h_ref[...]
    dv_ref[...] = jnp.broadcast_to(dinv, (BN, 16))


def _scale(d0, d1, h):
    return pl.pallas_call(
        _scale_body,
        grid=(GRID,),
        in_specs=[
            pl.BlockSpec((BN, F_PAD), lambda i: (i, 0)),
            pl.BlockSpec((BN, F_PAD), lambda i: (i, 0)),
            pl.BlockSpec((BN, F_PAD), lambda i: (i, 0)),
        ],
        out_specs=[
            pl.BlockSpec((BN, F_PAD), lambda i: (i, 0)),
            pl.BlockSpec((BN, 16), lambda i: (i, 0)),
        ],
        out_shape=[
            jax.ShapeDtypeStruct((N_PAD, F_PAD), jnp.float32),
            jax.ShapeDtypeStruct((N_PAD, 16), jnp.float32),
        ],
    )(d0, d1, h)


def _dot(a, b):
    return lax.dot_general(a, b, (((1,), (0,)), ((), ())),
                           preferred_element_type=jnp.float32)


def _gru_body(p0_ref, p1_ref, g_ref, dv_ref, h_ref, bc_ref,
              lt_ref, lb_ref, lbias_ref, lw_ref, lb2_ref,
              y_ref, hn_ref):
    dinv = dv_ref[:, 0:1]
    conv = dinv * (p0_ref[:, 0:F_ALL] + p1_ref[:, 0:F_ALL]
                   + g_ref[:, 0:F_ALL]) + bc_ref[...]
    H = h_ref[...]
    cz = conv[:, 0:32]
    cr = conv[:, 32:64]
    ch = conv[:, 64:96]
    lt = lt_ref[...]
    lb = lb_ref[...]
    lbias = lbias_ref[...]
    Z = jax.nn.sigmoid(_dot(cz, lt[:, 0:32]) + _dot(H, lb[:, 0:32])
                       + lbias[:, 0:32])
    R = jax.nn.sigmoid(_dot(cr, lt[:, 32:64]) + _dot(H, lb[:, 32:64])
                       + lbias[:, 32:64])
    Ht = jnp.tanh(_dot(ch, lt[:, 64:96]) + _dot(H * R, lb[:, 64:96])
                  + lbias[:, 64:96])
    Hn = Z * H + (1.0 - Z) * Ht
    hn_ref[...] = Hn
    y_ref[...] = _dot(jnp.maximum(Hn, 0.0), lw_ref[...]) + lb2_ref[...]


def _gru(p0, p1, g, dv, Hp, bcat, lt, lb, lbias, lin_w, lin_b):
    return pl.pallas_call(
        _gru_body,
        grid=(GRID,),
        in_specs=[
            pl.BlockSpec((BN, F_PAD), lambda i: (i, 0)),
            pl.BlockSpec((BN, F_PAD), lambda i: (i, 0)),
            pl.BlockSpec((BN, F_PAD), lambda i: (i, 0)),
            pl.BlockSpec((BN, 16), lambda i: (i, 0)),
            pl.BlockSpec((BN, F_OUT), lambda i: (i, 0)),
            pl.BlockSpec((1, F_ALL), lambda i: (0, 0)),
            pl.BlockSpec((F_OUT, F_ALL), lambda i: (0, 0)),
            pl.BlockSpec((F_OUT, F_ALL), lambda i: (0, 0)),
            pl.BlockSpec((1, F_ALL), lambda i: (0, 0)),
            pl.BlockSpec((F_OUT, 1), lambda i: (0, 0)),
            pl.BlockSpec((1, 1), lambda i: (0, 0)),
        ],
        out_specs=[
            pl.BlockSpec((BN, 1), lambda i: (i, 0)),
            pl.BlockSpec((BN, F_OUT), lambda i: (i, 0)),
        ],
        out_shape=[
            jax.ShapeDtypeStruct((N_PAD, 1), jnp.float32),
            jax.ShapeDtypeStruct((N_PAD, F_OUT), jnp.float32),
        ],
    )(p0, p1, g, dv, Hp, bcat, lt, lb, lbias, lin_w, lin_b)


def kernel(node_feat, edge_index, edge_weight, hidden_state,
           Wz, bz, Wr, br, Wh, bh,
           Lz_w, Lz_b, Lr_w, Lr_b, Lh_w, Lh_b, lin_w, lin_b):
    x = jnp.pad(node_feat, ((0, N_PAD - N), (0, 0)))
    w_all = jnp.pad(jnp.concatenate([Wz, Wr, Wh], axis=1),
                    ((0, 0), (0, F_PAD - F_ALL)))

    src = jnp.pad(edge_index[0], (0, E_PAD - E)).reshape(E_PAD // CH, CH)
    dst = jnp.pad(edge_index[1], (0, E_PAD - E)).reshape(E_PAD // CH, CH)
    w = jnp.pad(edge_weight, (0, E_PAD - E)).reshape(E_PAD // CH, CH)

    h = _matmul(x, w_all)
    degp = _sc_degree(dst, w)
    g, dv = _scale(degp[0], degp[1], h)
    accp = _sc_scatter(g, src, dst, w)

    Hp = jnp.pad(hidden_state, ((0, N_PAD - N), (0, 0)))
    bcat = jnp.concatenate([bz, br, bh])[None, :]
    lt = jnp.concatenate([Lz_w[:F_OUT], Lr_w[:F_OUT], Lh_w[:F_OUT]], axis=1)
    lb = jnp.concatenate([Lz_w[F_OUT:], Lr_w[F_OUT:], Lh_w[F_OUT:]], axis=1)
    lbias = jnp.concatenate([Lz_b, Lr_b, Lh_b])[None, :]

    y_p, hn_p = _gru(accp[0], accp[1], g, dv, Hp, bcat, lt, lb, lbias,
                     lin_w, lin_b[None, :])
    return (y_p[:N], hn_p[:N])
